# baseline (speedup 1.0000x reference)
import functools

import jax
import jax.numpy as jnp
from jax import lax
from jax.experimental import pallas as pl
from jax.experimental.pallas import tpu as pltpu
from jax.experimental.pallas import tpu_sc as plsc

N = 10000
E = 320000
D = 128
NC, NS = 2, 16
NW = NC * NS
EPT = E // NW
CH = 80
NCHUNK = EPT // CH
NBUF = 5
ROWS_PT = N // NS
HR = N // 16
HCH = 125


def _sc_segment_sum(x, srcr, dstr, zrows, zcnt, iot):
    mesh = plsc.VectorSubcoreMesh(core_axis_name="c", subcore_axis_name="s")

    @functools.partial(
        pl.kernel,
        mesh=mesh,
        out_type=(
            jax.ShapeDtypeStruct((NC, NS, ROWS_PT, D), jnp.bfloat16),
            jax.ShapeDtypeStruct((NC, HR, 16), jnp.float32),
        ),
        scratch_types=[
            pltpu.VMEM((NCHUNK, CH), jnp.int32),
            pltpu.VMEM((NCHUNK, CH), jnp.int32),
            pltpu.VMEM((NBUF, CH, D), jnp.bfloat16),
            pltpu.VMEM((HR, 16), jnp.float32),
            pltpu.VMEM((HR // HCH, HCH), jnp.int32),
            pltpu.VMEM_SHARED((N, D), jnp.bfloat16),
            pltpu.VMEM_SHARED((HR, 16), jnp.float32),
        ] + [pltpu.SemaphoreType.DMA] * (NBUF + 1),
        compiler_params=pltpu.CompilerParams(use_tc_tiling_on_sc=False,
                                             needs_layout_passes=False),
    )
    def k(x_hbm, src_hbm, dst_hbm, z_hbm, zc_hbm, iot_hbm, part_hbm,
          cnt_hbm, src_v, dst_v, rows, hist, iot_v, acc_sh, cnt_sh, *sems):
        zsem = sems[NBUF]
        c = lax.axis_index("c")
        s = lax.axis_index("s")
        wid = c * NS + s
        ones16 = jnp.ones((16,), jnp.float32)

        pltpu.async_copy(z_hbm, acc_sh.at[pl.ds(s * ROWS_PT, ROWS_PT)], zsem)
        pltpu.sync_copy(src_hbm.at[wid], src_v)
        pltpu.sync_copy(dst_hbm.at[wid], dst_v)
        pltpu.sync_copy(zc_hbm, hist)
        pltpu.sync_copy(iot_hbm, iot_v)

        @pl.when(s == 0)
        def _():
            pltpu.sync_copy(zc_hbm, cnt_sh)

        for b in range(NBUF):
            pltpu.async_copy(x_hbm.at[src_v.at[b]], rows.at[b], sems[b])
        pltpu.make_async_copy(z_hbm, acc_sh.at[pl.ds(s * ROWS_PT, ROWS_PT)],
                              zsem).wait()
        plsc.subcore_barrier()

        def group(g, carry):
            for b in range(NBUF):
                j = g * NBUF + b
                pltpu.make_async_copy(x_hbm.at[src_v.at[j]], rows.at[b],
                                      sems[b]).wait()
                pltpu.sync_copy(rows.at[b], acc_sh.at[dst_v.at[j]], add=True)

                @pl.when(j + NBUF < NCHUNK)
                def _():
                    pltpu.async_copy(x_hbm.at[src_v.at[j + NBUF]],
                                     rows.at[b], sems[b])

                for kk in range(CH // 16):
                    idx = dst_v[j, pl.ds(kk * 16, 16)]
                    r = lax.shift_right_logical(idx, 4)
                    q = lax.bitwise_and(idx, 15)
                    plsc.addupdate_scatter(hist, [r, q], ones16)
            return carry

        lax.fori_loop(0, NCHUNK // NBUF, group, 0)

        for kk in range(HR // HCH):
            pltpu.sync_copy(hist.at[pl.ds(kk * HCH, HCH)],
                            cnt_sh.at[iot_v.at[kk]], add=True)
        plsc.subcore_barrier()
        pltpu.sync_copy(acc_sh.at[pl.ds(s * ROWS_PT, ROWS_PT)],
                        part_hbm.at[c, s])

        @pl.when(s == 0)
        def _():
            pltpu.sync_copy(cnt_sh, cnt_hbm.at[c])

    return k(x, srcr, dstr, zrows, zcnt, iot)


def _tc_finish(parts, cnts, x, wl, wr, b):
    B = 1000

    def body(p_ref, c_ref, x_ref, wl_ref, wr_ref, b_ref, o_ref):
        p = p_ref[...].astype(jnp.float32)
        summed = p[0] + p[1]
        cnt = jnp.sum(c_ref[...], axis=1, keepdims=True)
        mean = summed / jnp.maximum(cnt, 1.0)
        dn = (((1,), (1,)), ((), ()))
        o_ref[...] = (
            lax.dot_general(mean, wl_ref[...], dn,
                            preferred_element_type=jnp.float32)
            + lax.dot_general(x_ref[...], wr_ref[...], dn,
                              preferred_element_type=jnp.float32)
            + b_ref[...]
        )

    return pl.pallas_call(
        body,
        grid=(N // B,),
        in_specs=[
            pl.BlockSpec((NC, B, D), lambda i: (0, i, 0)),
            pl.BlockSpec((B, NC), lambda i: (i, 0)),
            pl.BlockSpec((B, D), lambda i: (i, 0)),
            pl.BlockSpec((D, D), lambda i: (0, 0)),
            pl.BlockSpec((D, D), lambda i: (0, 0)),
            pl.BlockSpec((1, D), lambda i: (0, 0)),
        ],
        out_specs=pl.BlockSpec((B, D), lambda i: (i, 0)),
        out_shape=jax.ShapeDtypeStruct((N, D), jnp.float32),
    )(parts, cnts, x, wl, wr, b)


def kernel(x, edge_index, W_l, b_l, W_r, training):
    src = edge_index[0].astype(jnp.int32).reshape(NW, NCHUNK, CH)
    dst = edge_index[1].astype(jnp.int32).reshape(NW, NCHUNK, CH)
    zrows = jnp.zeros((ROWS_PT, D), jnp.bfloat16)
    zcnt = jnp.zeros((HR, 16), jnp.float32)
    iot = jnp.arange(HR, dtype=jnp.int32).reshape(HR // HCH, HCH)
    xb = x.astype(jnp.bfloat16)
    parts, cnts = _sc_segment_sum(xb, src, dst, zrows, zcnt, iot)
    parts = parts.reshape(NC, N, D)
    cnts = cnts.transpose(1, 2, 0).reshape(N, NC)
    return _tc_finish(parts, cnts, x, W_l, W_r, b_l.reshape(1, D))


# speedup vs baseline: 1.0202x; 1.0202x over previous
import functools

import jax
import jax.numpy as jnp
from jax import lax
from jax.experimental import pallas as pl
from jax.experimental.pallas import tpu as pltpu
from jax.experimental.pallas import tpu_sc as plsc

N = 10000
E = 320000
D = 128
NC, NS = 2, 16
NW = NC * NS
EPT = E // NW
CH = 80
NCHUNK = EPT // CH
NBUF = 5
ROWS_PT = N // NS


def _sc_segment_sum(x, srcr, dstr, zrows, zcnt):
    mesh = plsc.VectorSubcoreMesh(core_axis_name="c", subcore_axis_name="s")

    @functools.partial(
        pl.kernel,
        mesh=mesh,
        out_type=(
            jax.ShapeDtypeStruct((NC, NS, ROWS_PT, D), jnp.bfloat16),
            jax.ShapeDtypeStruct((NC, NS, N), jnp.float32),
        ),
        scratch_types=[
            pltpu.VMEM((NCHUNK, CH), jnp.int32),
            pltpu.VMEM((NCHUNK, CH), jnp.int32),
            pltpu.VMEM((NBUF, CH, D), jnp.bfloat16),
            pltpu.VMEM((N,), jnp.float32),
            pltpu.VMEM_SHARED((N, D), jnp.bfloat16),
        ] + [pltpu.SemaphoreType.DMA] * (NBUF + 1),
        compiler_params=pltpu.CompilerParams(use_tc_tiling_on_sc=False,
                                             needs_layout_passes=False),
    )
    def k(x_hbm, src_hbm, dst_hbm, z_hbm, zc_hbm, part_hbm, cnt_hbm,
          src_v, dst_v, rows, hist, acc_sh, *sems):
        zsem = sems[NBUF]
        c = lax.axis_index("c")
        s = lax.axis_index("s")
        wid = c * NS + s
        ones16 = jnp.ones((16,), jnp.float32)

        pltpu.async_copy(z_hbm, acc_sh.at[pl.ds(s * ROWS_PT, ROWS_PT)], zsem)
        pltpu.sync_copy(src_hbm.at[wid], src_v)
        pltpu.sync_copy(dst_hbm.at[wid], dst_v)
        pltpu.sync_copy(zc_hbm, hist)
        for b in range(NBUF):
            pltpu.async_copy(x_hbm.at[src_v.at[b]], rows.at[b], sems[b])
        pltpu.make_async_copy(z_hbm, acc_sh.at[pl.ds(s * ROWS_PT, ROWS_PT)],
                              zsem).wait()
        plsc.subcore_barrier()

        def group(g, carry):
            for b in range(NBUF):
                j = g * NBUF + b
                pltpu.make_async_copy(x_hbm.at[src_v.at[j]], rows.at[b],
                                      sems[b]).wait()
                pltpu.sync_copy(rows.at[b], acc_sh.at[dst_v.at[j]], add=True)

                @pl.when(j + NBUF < NCHUNK)
                def _():
                    pltpu.async_copy(x_hbm.at[src_v.at[j + NBUF]],
                                     rows.at[b], sems[b])

                for kk in range(CH // 16):
                    idx = dst_v[j, pl.ds(kk * 16, 16)]
                    plsc.addupdate_scatter(hist, [idx], ones16)
            return carry

        lax.fori_loop(0, NCHUNK // NBUF, group, 0)

        pltpu.sync_copy(hist, cnt_hbm.at[c, s])
        plsc.subcore_barrier()
        pltpu.sync_copy(acc_sh.at[pl.ds(s * ROWS_PT, ROWS_PT)],
                        part_hbm.at[c, s])

    return k(x, srcr, dstr, zrows, zcnt)


def _tc_finish(parts, cnts, x, wlt, wrt, b):
    B = 1000

    def body(p_ref, c_ref, x_ref, wlt_ref, wrt_ref, b_ref, o_ref):
        p = p_ref[...].astype(jnp.float32)
        summed = p[0] + p[1]
        cnt = jnp.sum(c_ref[...], axis=1, keepdims=True)
        mean = summed / jnp.maximum(cnt, 1.0)
        dn = (((1,), (1,)), ((), ()))
        o_ref[...] = (
            lax.dot_general(mean, wlt_ref[...], dn,
                            preferred_element_type=jnp.float32)
            + lax.dot_general(x_ref[...], wrt_ref[...], dn,
                              preferred_element_type=jnp.float32)
            + b_ref[...]
        )

    return pl.pallas_call(
        body,
        grid=(N // B,),
        in_specs=[
            pl.BlockSpec((NC, B, D), lambda i: (0, i, 0)),
            pl.BlockSpec((B, NW), lambda i: (i, 0)),
            pl.BlockSpec((B, D), lambda i: (i, 0)),
            pl.BlockSpec((D, D), lambda i: (0, 0)),
            pl.BlockSpec((D, D), lambda i: (0, 0)),
            pl.BlockSpec((1, D), lambda i: (0, 0)),
        ],
        out_specs=pl.BlockSpec((B, D), lambda i: (i, 0)),
        out_shape=jax.ShapeDtypeStruct((N, D), jnp.float32),
    )(parts, cnts, x, wlt, wrt, b)


def kernel(x, edge_index, W_l, b_l, W_r, training):
    src = edge_index[0].astype(jnp.int32).reshape(NW, NCHUNK, CH)
    dst = edge_index[1].astype(jnp.int32).reshape(NW, NCHUNK, CH)
    zrows = jnp.zeros((ROWS_PT, D), jnp.bfloat16)
    zcnt = jnp.zeros((N,), jnp.float32)
    xb = x.astype(jnp.bfloat16)
    parts, cnts = _sc_segment_sum(xb, src, dst, zrows, zcnt)
    parts = parts.reshape(NC, N, D)
    cnts = cnts.reshape(NW, N).T
    return _tc_finish(parts, cnts, x, W_l, W_r, b_l.reshape(1, D))


# speedup vs baseline: 1.0396x; 1.0190x over previous
import functools

import jax
import jax.numpy as jnp
from jax import lax
from jax.experimental import pallas as pl
from jax.experimental.pallas import tpu as pltpu
from jax.experimental.pallas import tpu_sc as plsc

N = 10000
E = 320000
D = 128
NC, NS = 2, 16
NW = NC * NS
EPT = E // NW
CH = 80
NCHUNK = EPT // CH
NBUF = 5
ROWS_PT = N // NS


def _sc_segment_sum(x, srcr, dstr, zrows, zcnt):
    mesh = plsc.VectorSubcoreMesh(core_axis_name="c", subcore_axis_name="s")

    @functools.partial(
        pl.kernel,
        mesh=mesh,
        out_type=(
            jax.ShapeDtypeStruct((NC, NS, ROWS_PT, D), jnp.bfloat16),
            jax.ShapeDtypeStruct((NC, NS, N), jnp.float32),
        ),
        scratch_types=[
            pltpu.VMEM((NCHUNK, CH), jnp.int32),
            pltpu.VMEM((NCHUNK, CH), jnp.int32),
            pltpu.VMEM((NBUF, CH, D), jnp.bfloat16),
            pltpu.VMEM((N,), jnp.float32),
            pltpu.VMEM_SHARED((N, D), jnp.bfloat16),
        ] + [pltpu.SemaphoreType.DMA] * (NBUF + 1),
        compiler_params=pltpu.CompilerParams(use_tc_tiling_on_sc=False,
                                             needs_layout_passes=False),
    )
    def k(x_hbm, src_hbm, dst_hbm, z_hbm, zc_hbm, part_hbm, cnt_hbm,
          src_v, dst_v, rows, hist, acc_sh, *sems):
        zsem = sems[NBUF]
        c = lax.axis_index("c")
        s = lax.axis_index("s")
        wid = c * NS + s
        ones16 = jnp.ones((16,), jnp.float32)

        pltpu.async_copy(z_hbm, acc_sh.at[pl.ds(s * ROWS_PT, ROWS_PT)], zsem)
        pltpu.sync_copy(src_hbm.at[wid], src_v)
        pltpu.sync_copy(dst_hbm.at[wid], dst_v)
        pltpu.sync_copy(zc_hbm, hist)
        for b in range(NBUF):
            pltpu.async_copy(x_hbm.at[src_v.at[b]], rows.at[b], sems[b])
        pltpu.make_async_copy(z_hbm, acc_sh.at[pl.ds(s * ROWS_PT, ROWS_PT)],
                              zsem).wait()
        plsc.subcore_barrier()

        def group(g, carry):
            for b in range(NBUF):
                j = g * NBUF + b
                pltpu.make_async_copy(x_hbm.at[src_v.at[j]], rows.at[b],
                                      sems[b]).wait()
                pltpu.sync_copy(rows.at[b], acc_sh.at[dst_v.at[j]], add=True)

                @pl.when(j + NBUF < NCHUNK)
                def _():
                    pltpu.async_copy(x_hbm.at[src_v.at[j + NBUF]],
                                     rows.at[b], sems[b])

                for kk in range(CH // 16):
                    idx = dst_v[j, pl.ds(kk * 16, 16)]
                    plsc.addupdate_scatter(hist, [idx], ones16)
            return carry

        lax.fori_loop(0, NCHUNK // NBUF, group, 0)

        pltpu.sync_copy(hist, cnt_hbm.at[c, s])
        plsc.subcore_barrier()
        pltpu.sync_copy(acc_sh.at[pl.ds(s * ROWS_PT, ROWS_PT)],
                        part_hbm.at[c, s])

    return k(x, srcr, dstr, zrows, zcnt)


def _tc_finish(parts, cnts, x, wlt, wrt, b):
    B = 2000

    def body(p_ref, c_ref, x_ref, wlt_ref, wrt_ref, b_ref, o_ref):
        p = p_ref[...].astype(jnp.float32)
        summed = p[0] + p[1]
        cnt = jnp.sum(c_ref[...], axis=1, keepdims=True)
        mean = summed / jnp.maximum(cnt, 1.0)
        dn = (((1,), (1,)), ((), ()))
        o_ref[...] = (
            lax.dot_general(mean, wlt_ref[...], dn,
                            preferred_element_type=jnp.float32)
            + lax.dot_general(x_ref[...], wrt_ref[...], dn,
                              preferred_element_type=jnp.float32)
            + b_ref[...]
        )

    return pl.pallas_call(
        body,
        grid=(N // B,),
        in_specs=[
            pl.BlockSpec((NC, B, D), lambda i: (0, i, 0)),
            pl.BlockSpec((B, NW), lambda i: (i, 0)),
            pl.BlockSpec((B, D), lambda i: (i, 0)),
            pl.BlockSpec((D, D), lambda i: (0, 0)),
            pl.BlockSpec((D, D), lambda i: (0, 0)),
            pl.BlockSpec((1, D), lambda i: (0, 0)),
        ],
        out_specs=pl.BlockSpec((B, D), lambda i: (i, 0)),
        out_shape=jax.ShapeDtypeStruct((N, D), jnp.float32),
    )(parts, cnts, x, wlt, wrt, b)


def kernel(x, edge_index, W_l, b_l, W_r, training):
    src = edge_index[0].astype(jnp.int32).reshape(NW, NCHUNK, CH)
    dst = edge_index[1].astype(jnp.int32).reshape(NW, NCHUNK, CH)
    zrows = jnp.zeros((ROWS_PT, D), jnp.bfloat16)
    zcnt = jnp.zeros((N,), jnp.float32)
    xb = x.astype(jnp.bfloat16)
    parts, cnts = _sc_segment_sum(xb, src, dst, zrows, zcnt)
    parts = parts.reshape(NC, N, D)
    cnts = cnts.reshape(NW, N).T
    return _tc_finish(parts, cnts, xb, W_l, W_r, b_l.reshape(1, D))
